# baseline (speedup 1.0000x reference)
import jax
import jax.numpy as jnp
from jax.experimental import pallas as pl
from jax.experimental.pallas import tpu as pltpu

N = 10000
D_IN = 256
D_OUT = 256

BM = 200
NB = N // BM
NBUF = 6


def _adj_copy(adj_hbm, adj_buf, sems, blk, slot):
    return pltpu.make_async_copy(
        adj_hbm.at[pl.ds(blk * BM, BM), :],
        adj_buf.at[slot],
        sems.at[slot],
    )


def _fused_kernel(x_ref, w_ref, adj_hbm, o_ref, adj_buf, sems):
    i = pl.program_id(0)

    @pl.when(i == 0)
    def _():
        for b in range(NBUF - 1):
            _adj_copy(adj_hbm, adj_buf, sems, b, b).start()

    nxt = i + NBUF - 1

    @pl.when(nxt < NB)
    def _():
        _adj_copy(adj_hbm, adj_buf, sems, nxt, jax.lax.rem(nxt, NBUF)).start()

    slot = jax.lax.rem(i, NBUF)
    _adj_copy(adj_hbm, adj_buf, sems, i, slot).wait()
    t = jnp.dot(
        adj_buf[slot],
        x_ref[...],
        precision=jax.lax.Precision.DEFAULT,
        preferred_element_type=jnp.float32,
    )
    o_ref[...] = jnp.dot(
        t,
        w_ref[...],
        precision=jax.lax.Precision.DEFAULT,
        preferred_element_type=jnp.float32,
    )


def kernel(input, adj, origin_features, weight, weight2):
    out = pl.pallas_call(
        _fused_kernel,
        grid=(NB,),
        in_specs=[
            pl.BlockSpec((N, D_IN), lambda i: (0, 0)),
            pl.BlockSpec((D_IN, D_OUT), lambda i: (0, 0)),
            pl.BlockSpec(memory_space=pl.ANY),
        ],
        out_specs=pl.BlockSpec((BM, D_OUT), lambda i: (i, 0)),
        out_shape=jax.ShapeDtypeStruct((N, D_OUT), jnp.float32),
        scratch_shapes=[
            pltpu.VMEM((NBUF, BM, N), jnp.float32),
            pltpu.SemaphoreType.DMA((NBUF,)),
        ],
    )(input, weight, adj)
    return out


# speedup vs baseline: 1.0013x; 1.0013x over previous
import jax
import jax.numpy as jnp
from jax.experimental import pallas as pl
from jax.experimental.pallas import tpu as pltpu

N = 10000
D_IN = 256
D_OUT = 256

BM = 200
NB = N // BM
NBUF = 4


def _adj_copy(adj_hbm, adj_buf, sems, blk, slot):
    return pltpu.make_async_copy(
        adj_hbm.at[pl.ds(blk * BM, BM), :],
        adj_buf.at[slot],
        sems.at[slot],
    )


def _fused_kernel(w_ref, x_hbm, adj_hbm, o_ref, s_ref, x_buf, adj_buf,
                  sems, x_sem):
    i = pl.program_id(0)

    @pl.when(i == 0)
    def _():
        for b in range(NBUF - 1):
            _adj_copy(adj_hbm, adj_buf, sems, b, b).start()
        x_cp = pltpu.make_async_copy(x_hbm, x_buf, x_sem)
        x_cp.start()
        x_cp.wait()
        s_ref[...] = jnp.dot(
            x_buf[...],
            w_ref[...],
            precision=jax.lax.Precision.DEFAULT,
            preferred_element_type=jnp.float32,
        )

    nxt = i + NBUF - 1

    @pl.when(nxt < NB)
    def _():
        _adj_copy(adj_hbm, adj_buf, sems, nxt, jax.lax.rem(nxt, NBUF)).start()

    slot = jax.lax.rem(i, NBUF)
    _adj_copy(adj_hbm, adj_buf, sems, i, slot).wait()
    o_ref[...] = jnp.dot(
        adj_buf[slot],
        s_ref[...],
        precision=jax.lax.Precision.DEFAULT,
        preferred_element_type=jnp.float32,
    )


def kernel(input, adj, origin_features, weight, weight2):
    out = pl.pallas_call(
        _fused_kernel,
        grid=(NB,),
        in_specs=[
            pl.BlockSpec((D_IN, D_OUT), lambda i: (0, 0)),
            pl.BlockSpec(memory_space=pl.ANY),
            pl.BlockSpec(memory_space=pl.ANY),
        ],
        out_specs=pl.BlockSpec((BM, D_OUT), lambda i: (i, 0)),
        out_shape=jax.ShapeDtypeStruct((N, D_OUT), jnp.float32),
        scratch_shapes=[
            pltpu.VMEM((N, D_OUT), jnp.float32),
            pltpu.VMEM((N, D_IN), jnp.float32),
            pltpu.VMEM((NBUF, BM, N), jnp.float32),
            pltpu.SemaphoreType.DMA((NBUF,)),
            pltpu.SemaphoreType.DMA,
        ],
    )(weight, input, adj)
    return out


# speedup vs baseline: 1.0196x; 1.0183x over previous
import jax
import jax.numpy as jnp
from jax.experimental import pallas as pl
from jax.experimental.pallas import tpu as pltpu

N = 10000
D_IN = 256
D_OUT = 256

BM = 200
NB = N // BM
NBUF = 4


def _adj_copy(adj_hbm, adj_buf, sems, blk, slot):
    return pltpu.make_async_copy(
        adj_hbm.at[pl.ds(blk * BM, BM), :],
        adj_buf.at[slot],
        sems.at[slot],
    )


def _fused_kernel(x_ref, w_ref, adj_hbm, o_ref, s_ref, adj_buf, sems):
    i = pl.program_id(0)

    @pl.when(i == 0)
    def _():
        for b in range(NBUF - 1):
            _adj_copy(adj_hbm, adj_buf, sems, b, b).start()
        s_ref[...] = jnp.dot(
            x_ref[...],
            w_ref[...],
            precision=jax.lax.Precision.DEFAULT,
            preferred_element_type=jnp.float32,
        )

    nxt = i + NBUF - 1

    @pl.when(nxt < NB)
    def _():
        _adj_copy(adj_hbm, adj_buf, sems, nxt, jax.lax.rem(nxt, NBUF)).start()

    slot = jax.lax.rem(i, NBUF)
    _adj_copy(adj_hbm, adj_buf, sems, i, slot).wait()
    o_ref[...] = jnp.dot(
        adj_buf[slot],
        s_ref[...],
        precision=jax.lax.Precision.DEFAULT,
        preferred_element_type=jnp.float32,
    )


def kernel(input, adj, origin_features, weight, weight2):
    out = pl.pallas_call(
        _fused_kernel,
        grid=(NB,),
        in_specs=[
            pl.BlockSpec((N, D_IN), lambda i: (0, 0)),
            pl.BlockSpec((D_IN, D_OUT), lambda i: (0, 0)),
            pl.BlockSpec(memory_space=pl.ANY),
        ],
        out_specs=pl.BlockSpec((BM, D_OUT), lambda i: (i, 0)),
        out_shape=jax.ShapeDtypeStruct((N, D_OUT), jnp.float32),
        scratch_shapes=[
            pltpu.VMEM((N, D_OUT), jnp.float32),
            pltpu.VMEM((NBUF, BM, N), jnp.float32),
            pltpu.SemaphoreType.DMA((NBUF,)),
        ],
    )(input, weight, adj)
    return out
